# Initial kernel scaffold; baseline (speedup 1.0000x reference)
import jax
import jax.numpy as jnp
from jax.experimental import pallas as pl


def kernel(x, ind, W, b):
    raise NotImplementedError("write your pallas kernel here")


# speedup vs baseline: 160.2503x; 160.2503x over previous
import jax
import jax.numpy as jnp
from jax.experimental import pallas as pl

T, DI, DO, E = 8192, 2048, 2048, 8
BT = 256
NT = T // BT


def _mm_body(ind_ref, x_ref, w_ref, b_ref, o_ref):
    e = pl.program_id(1)

    @pl.when(e == 0)
    def _init():
        o_ref[...] = jnp.zeros_like(o_ref)

    m = ind_ref[0] == e
    xm = jnp.where(m, x_ref[...], 0.0)
    acc = jax.lax.dot_general(xm, w_ref[0], (((1,), (1,)), ((), ())),
                              preferred_element_type=jnp.float32)
    o_ref[...] += acc + jnp.where(m, b_ref[0], 0.0)


def kernel(x, ind, W, b):
    ind3 = ind.reshape(NT, BT, 1)
    b3 = b.reshape(E, 1, DO)
    return pl.pallas_call(
        _mm_body,
        grid=(NT, E),
        in_specs=[
            pl.BlockSpec((1, BT, 1), lambda t, e: (t, 0, 0)),
            pl.BlockSpec((BT, DI), lambda t, e: (t, 0)),
            pl.BlockSpec((1, DO, DI), lambda t, e: (e, 0, 0)),
            pl.BlockSpec((1, 1, DO), lambda t, e: (e, 0, 0)),
        ],
        out_specs=pl.BlockSpec((BT, DO), lambda t, e: (t, 0)),
        out_shape=jax.ShapeDtypeStruct((T, DO), jnp.float32),
    )(ind3, x, W, b3)


# speedup vs baseline: 362.1197x; 2.2597x over previous
import jax
import jax.numpy as jnp
from jax.experimental import pallas as pl
from jax.experimental.pallas import tpu as pltpu

T, DI, DO, E = 8192, 2048, 2048, 8
BT = 256
NPAD = T + E * BT
NB = NPAD // BT


def _routing(ind):
    i32 = jnp.int32
    counts = jnp.bincount(ind, length=E).astype(i32)
    padded = (counts + BT - 1) // BT * BT
    cpe = jnp.cumsum(padded)
    poff = cpe - padded
    oh = (ind[:, None] == jnp.arange(E, dtype=i32)[None, :]).astype(i32)
    rank = jnp.take_along_axis(jnp.cumsum(oh, 0), ind[:, None], 1)[:, 0] - 1
    pos = poff[ind] + rank
    src = jnp.zeros(NPAD, i32).at[pos].set(jnp.arange(T, dtype=i32))
    blk_start = jnp.arange(NB, dtype=i32) * BT
    block_expert = jnp.sum(blk_start[:, None] >= cpe[None, :], axis=1)
    block_expert = jnp.minimum(block_expert, E - 1).astype(i32)
    nb_real = (cpe[-1] // BT).astype(i32).reshape(1)
    return pos, src, block_expert, nb_real


def _gemm_body(be_ref, nbr_ref, xs_ref, w_ref, b_ref, ys_ref):
    @pl.when(pl.program_id(0) < nbr_ref[0])
    def _():
        acc = jax.lax.dot_general(xs_ref[...], w_ref[0],
                                  (((1,), (1,)), ((), ())),
                                  preferred_element_type=jnp.float32)
        ys_ref[...] = acc + b_ref[0]


def kernel(x, ind, W, b):
    pos, src, block_expert, nb_real = _routing(ind)
    xs = jnp.take(x, src, axis=0)
    b3 = b.reshape(E, 1, DO)

    grid_spec = pltpu.PrefetchScalarGridSpec(
        num_scalar_prefetch=2,
        grid=(NB,),
        in_specs=[
            pl.BlockSpec((BT, DI), lambda g, be, nbr: (g, 0)),
            pl.BlockSpec((1, DO, DI), lambda g, be, nbr: (be[g], 0, 0)),
            pl.BlockSpec((1, 1, DO), lambda g, be, nbr: (be[g], 0, 0)),
        ],
        out_specs=pl.BlockSpec((BT, DO), lambda g, be, nbr: (g, 0)),
    )
    ys = pl.pallas_call(
        _gemm_body,
        grid_spec=grid_spec,
        out_shape=jax.ShapeDtypeStruct((NPAD, DO), jnp.float32),
    )(block_expert, nb_real, xs, W, b3)
    return jnp.take(ys, pos, axis=0)


# speedup vs baseline: 373.9382x; 1.0326x over previous
import jax
import jax.numpy as jnp
from jax.experimental import pallas as pl
from jax.experimental.pallas import tpu as pltpu

T, DI, DO, E = 8192, 2048, 2048, 8
BT = 256
NPAD = T + E * BT
NB = NPAD // BT


def _routing(ind):
    i32 = jnp.int32
    oh = (ind[:, None] == jnp.arange(E, dtype=i32)[None, :]).astype(i32)
    counts = jnp.sum(oh, axis=0)
    padded = (counts + BT - 1) // BT * BT
    cpe = jnp.cumsum(padded)
    poff = cpe - padded
    pos = jnp.sum(oh * (jnp.cumsum(oh, 0) - 1 + poff[None, :]), axis=1)
    src = jnp.zeros(NPAD, i32).at[pos].set(jnp.arange(T, dtype=i32))
    blk_start = jnp.arange(NB, dtype=i32) * BT
    block_expert = jnp.sum(blk_start[:, None] >= cpe[None, :], axis=1)
    block_expert = jnp.minimum(block_expert, E - 1).astype(i32)
    nb_real = (cpe[-1] // BT).astype(i32).reshape(1)
    return pos, src, block_expert, nb_real


def _gemm_body(be_ref, nbr_ref, xs_ref, w_ref, b_ref, ys_ref):
    @pl.when(pl.program_id(0) < nbr_ref[0])
    def _():
        acc = jax.lax.dot_general(xs_ref[...], w_ref[0],
                                  (((1,), (1,)), ((), ())),
                                  preferred_element_type=jnp.float32)
        ys_ref[...] = acc + b_ref[0]


def kernel(x, ind, W, b):
    pos, src, block_expert, nb_real = _routing(ind)
    xs = jnp.take(x, src, axis=0)
    b3 = b.reshape(E, 1, DO)

    grid_spec = pltpu.PrefetchScalarGridSpec(
        num_scalar_prefetch=2,
        grid=(NB,),
        in_specs=[
            pl.BlockSpec((BT, DI), lambda g, be, nbr: (g, 0)),
            pl.BlockSpec((1, DO, DI), lambda g, be, nbr: (be[g], 0, 0)),
            pl.BlockSpec((1, 1, DO), lambda g, be, nbr: (be[g], 0, 0)),
        ],
        out_specs=pl.BlockSpec((BT, DO), lambda g, be, nbr: (g, 0)),
    )
    ys = pl.pallas_call(
        _gemm_body,
        grid_spec=grid_spec,
        out_shape=jax.ShapeDtypeStruct((NPAD, DO), jnp.float32),
    )(block_expert, nb_real, xs, W, b3)
    return jnp.take(ys, pos, axis=0)


# speedup vs baseline: 453.1774x; 1.2119x over previous
import functools

import jax
import jax.numpy as jnp
from jax import lax
from jax.experimental import pallas as pl
from jax.experimental.pallas import tpu as pltpu
from jax.experimental.pallas import tpu_sc as plsc

T, DI, DO, E = 8192, 2048, 2048, 8
BT = 256
NPAD = T + E * BT
NB = NPAD // BT

NW = 32
CH = 16


def _make_sc_row_gather(n_rows, n_cols):
    mpw = n_rows // NW
    nch = mpw // CH
    mesh = plsc.VectorSubcoreMesh(core_axis_name="c", subcore_axis_name="s")

    @functools.partial(
        pl.kernel, mesh=mesh,
        out_type=jax.ShapeDtypeStruct((n_rows, n_cols), jnp.float32),
        scratch_types=[
            pltpu.VMEM((nch, CH), jnp.int32),
            pltpu.VMEM((CH, n_cols), jnp.float32),
            pltpu.VMEM((CH, n_cols), jnp.float32),
            pltpu.SemaphoreType.DMA,
            pltpu.SemaphoreType.DMA,
        ],
    )
    def gather_k(table_hbm, idx_hbm, out_hbm, idx_v, buf0, buf1, sem0, sem1):
        wid = lax.axis_index("s") * 2 + lax.axis_index("c")
        base = wid * mpw
        pltpu.sync_copy(idx_hbm.at[wid], idx_v)
        bufs = (buf0, buf1)
        sems = (sem0, sem1)

        handles = [None, None]
        handles[0] = pltpu.async_copy(table_hbm.at[idx_v.at[0]], buf0, sem0)
        for c in range(nch):
            slot = c % 2
            handles[slot].wait()
            if c + 1 < nch:
                nslot = (c + 1) % 2
                handles[nslot] = pltpu.async_copy(
                    table_hbm.at[idx_v.at[c + 1]], bufs[nslot], sems[nslot])
            pltpu.sync_copy(bufs[slot], out_hbm.at[pl.ds(base + c * CH, CH)])

    def run(table, idx):
        return gather_k(table, idx.reshape(NW, nch, CH))

    return run


_sc_gather_npad = _make_sc_row_gather(NPAD, DI)
_sc_gather_t = _make_sc_row_gather(T, DO)


def _routing(ind):
    i32 = jnp.int32
    oh = (ind[:, None] == jnp.arange(E, dtype=i32)[None, :]).astype(i32)
    counts = jnp.sum(oh, axis=0)
    padded = (counts + BT - 1) // BT * BT
    cpe = jnp.cumsum(padded)
    poff = cpe - padded
    pos = jnp.sum(oh * (jnp.cumsum(oh, 0) - 1 + poff[None, :]), axis=1)
    src = jnp.zeros(NPAD, i32).at[pos].set(jnp.arange(T, dtype=i32))
    blk_start = jnp.arange(NB, dtype=i32) * BT
    block_expert = jnp.sum(blk_start[:, None] >= cpe[None, :], axis=1)
    block_expert = jnp.minimum(block_expert, E - 1).astype(i32)
    nb_real = (cpe[-1] // BT).astype(i32).reshape(1)
    return pos, src, block_expert, nb_real


def _gemm_body(be_ref, nbr_ref, xs_ref, w_ref, b_ref, ys_ref):
    @pl.when(pl.program_id(0) < nbr_ref[0])
    def _():
        acc = jax.lax.dot_general(xs_ref[...], w_ref[0],
                                  (((1,), (1,)), ((), ())),
                                  preferred_element_type=jnp.float32)
        ys_ref[...] = acc + b_ref[0]


def kernel(x, ind, W, b):
    pos, src, block_expert, nb_real = _routing(ind)
    xs = _sc_gather_npad(x, src)
    b3 = b.reshape(E, 1, DO)

    grid_spec = pltpu.PrefetchScalarGridSpec(
        num_scalar_prefetch=2,
        grid=(NB,),
        in_specs=[
            pl.BlockSpec((BT, DI), lambda g, be, nbr: (g, 0)),
            pl.BlockSpec((1, DO, DI), lambda g, be, nbr: (be[g], 0, 0)),
            pl.BlockSpec((1, 1, DO), lambda g, be, nbr: (be[g], 0, 0)),
        ],
        out_specs=pl.BlockSpec((BT, DO), lambda g, be, nbr: (g, 0)),
    )
    ys = pl.pallas_call(
        _gemm_body,
        grid_spec=grid_spec,
        out_shape=jax.ShapeDtypeStruct((NPAD, DO), jnp.float32),
    )(block_expert, nb_real, xs, W, b3)
    return _sc_gather_t(ys, pos)
